# Initial kernel scaffold; baseline (speedup 1.0000x reference)
import jax
import jax.numpy as jnp
from jax.experimental import pallas as pl


def kernel(hidden_states, topk_weights, topk_ids, gate_up_weight, down_weight):
    raise NotImplementedError("write your pallas kernel here")


# speedup vs baseline: 1.4653x; 1.4653x over previous
import functools

import jax
import jax.numpy as jnp
from jax.experimental import pallas as pl
from jax.experimental.pallas import tpu as pltpu

NUM_EXPERTS_C = 8
TOP_K_C = 2
HIDDEN_C = 1024
FFN_C = 4096
TOKENS_C = 2048
BN = 512


def _moe_dense_body(x_ref, tw_ref, ids_ref, wg_ref, wu_ref, wd_ref,
                    out_ref, xbf_ref):
    e = pl.program_id(0)
    n = pl.program_id(1)

    @pl.when(jnp.logical_and(e == 0, n == 0))
    def _init():
        xbf_ref[...] = x_ref[...].astype(jnp.bfloat16)
        out_ref[...] = jnp.zeros_like(out_ref)

    xbf = xbf_ref[...]
    wg = wg_ref[0].astype(jnp.bfloat16)
    wu = wu_ref[0].astype(jnp.bfloat16)
    g = jax.lax.dot_general(xbf, wg, (((1,), (1,)), ((), ())),
                            preferred_element_type=jnp.float32)
    u = jax.lax.dot_general(xbf, wu, (((1,), (1,)), ((), ())),
                            preferred_element_type=jnp.float32)
    h = (g * jax.lax.logistic(g)) * u

    mask = (ids_ref[...] == e).astype(jnp.float32)
    we = jnp.sum(tw_ref[...] * mask, axis=1)
    hw = (h * we[:, None]).astype(jnp.bfloat16)

    wd = wd_ref[0].astype(jnp.bfloat16)
    part = jax.lax.dot_general(hw, wd, (((1,), (1,)), ((), ())),
                               preferred_element_type=jnp.float32)
    out_ref[...] += part


@jax.jit
def kernel(hidden_states, topk_weights, topk_ids, gate_up_weight, down_weight):
    T, H = hidden_states.shape
    E = gate_up_weight.shape[0]
    F = down_weight.shape[2]
    n_blocks = F // BN

    grid = (E, n_blocks)
    out = pl.pallas_call(
        _moe_dense_body,
        grid=grid,
        in_specs=[
            pl.BlockSpec((T, H), lambda e, n: (0, 0)),
            pl.BlockSpec((T, TOP_K_C), lambda e, n: (0, 0)),
            pl.BlockSpec((T, TOP_K_C), lambda e, n: (0, 0)),
            pl.BlockSpec((1, BN, H), lambda e, n: (e, n, 0)),
            pl.BlockSpec((1, BN, H), lambda e, n: (e, n + FFN_C // BN, 0)),
            pl.BlockSpec((1, H, BN), lambda e, n: (e, 0, n)),
        ],
        out_specs=pl.BlockSpec((T, H), lambda e, n: (0, 0)),
        out_shape=jax.ShapeDtypeStruct((T, H), jnp.float32),
        scratch_shapes=[pltpu.VMEM((T, H), jnp.bfloat16)],
    )(hidden_states, topk_weights, topk_ids,
      gate_up_weight, gate_up_weight, down_weight)
    return out
